# Initial kernel scaffold; baseline (speedup 1.0000x reference)
import jax
import jax.numpy as jnp
from jax.experimental import pallas as pl


def kernel(pred_deltas, pred_label_probs, prior_boxes):
    raise NotImplementedError("write your pallas kernel here")


# speedup vs baseline: 8.6870x; 8.6870x over previous
import functools

import jax
import jax.numpy as jnp
from jax import lax
from jax.experimental import pallas as pl
from jax.experimental.pallas import tpu as pltpu

_N = 20000
_NPAD = 20480
_NBLK = 10240
_C = 21
_CPAD = 24
_B = 8
_PRE = 600
_PREPAD = 640
_INST = _B * _C
_LANES = 256
_MAXT = 200
_SCORE_TH = 0.5
_IOU_TH = 0.5


def _decode_body(deltas_ref, priors_ref, probs_ref, boxes_ref, scores_ref):
    d = deltas_ref[0]
    p = priors_ref[...]
    pw = p[3:4] - p[1:2]
    ph = p[2:3] - p[0:1]
    pcx = p[1:2] + 0.5 * pw
    pcy = p[0:1] + 0.5 * ph
    bw = jnp.exp(d[3:4] * 0.2) * pw
    bh = jnp.exp(d[2:3] * 0.2) * ph
    bcx = (d[1:2] * 0.1) * pw + pcx
    bcy = (d[0:1] * 0.1) * ph + pcy
    y1 = bcy - 0.5 * bh
    x1 = bcx - 0.5 * bw
    y2 = bh + y1
    x2 = bw + x1
    boxes_ref[0] = jnp.clip(jnp.concatenate([y1, x1, y2, x2], axis=0), 0.0, 1.0)
    pr = probs_ref[0]
    mx = jnp.max(pr, axis=0, keepdims=True)
    nonbg = pr[0:1] < mx
    scores_ref[0] = jnp.where(nonbg, pr, 0.0)


def _decode(pred_deltas, pred_label_probs, prior_boxes):
    deltas_t = jnp.pad(jnp.swapaxes(pred_deltas, 1, 2), ((0, 0), (0, 0), (0, _NPAD - _N)))
    priors_t = jnp.pad(prior_boxes.T, ((0, 0), (0, _NPAD - _N)))
    probs_t = jnp.pad(
        jnp.swapaxes(pred_label_probs, 1, 2),
        ((0, 0), (0, _CPAD - _C), (0, _NPAD - _N)),
        constant_values=-1.0,
    )
    grid = (_B, _NPAD // _NBLK)
    boxes_t, scores_t = pl.pallas_call(
        _decode_body,
        grid=grid,
        in_specs=[
            pl.BlockSpec((1, 4, _NBLK), lambda b, n: (b, 0, n)),
            pl.BlockSpec((4, _NBLK), lambda b, n: (0, n)),
            pl.BlockSpec((1, _CPAD, _NBLK), lambda b, n: (b, 0, n)),
        ],
        out_specs=[
            pl.BlockSpec((1, 4, _NBLK), lambda b, n: (b, 0, n)),
            pl.BlockSpec((1, _CPAD, _NBLK), lambda b, n: (b, 0, n)),
        ],
        out_shape=[
            jax.ShapeDtypeStruct((_B, 4, _NPAD), jnp.float32),
            jax.ShapeDtypeStruct((_B, _CPAD, _NPAD), jnp.float32),
        ],
    )(deltas_t, priors_t, probs_t)
    return boxes_t, scores_t


def _nms_body(y1_ref, x1_ref, y2_ref, x2_ref, sc_ref, out_ref, area_ref, keep_ref):
    y1 = y1_ref[...]
    x1 = x1_ref[...]
    y2 = y2_ref[...]
    x2 = x2_ref[...]
    area_ref[...] = (y2 - y1) * (x2 - x1)
    keep_ref[...] = jnp.ones_like(y1)
    rows = lax.broadcasted_iota(jnp.int32, (_PREPAD, _LANES), 0)

    def body(i, _):
        keep = keep_ref[...]
        ry1 = y1_ref[pl.ds(i, 1), :]
        rx1 = x1_ref[pl.ds(i, 1), :]
        ry2 = y2_ref[pl.ds(i, 1), :]
        rx2 = x2_ref[pl.ds(i, 1), :]
        rsc = sc_ref[pl.ds(i, 1), :]
        rkeep = keep_ref[pl.ds(i, 1), :]
        rarea = area_ref[pl.ds(i, 1), :]
        can = (rkeep > 0.0) & (rsc > _SCORE_TH)
        ih = jnp.maximum(jnp.minimum(y2, ry2) - jnp.maximum(y1, ry1), 0.0)
        iw = jnp.maximum(jnp.minimum(x2, rx2) - jnp.maximum(x1, rx1), 0.0)
        inter = ih * iw
        union = area_ref[...] + rarea - inter
        iou = inter / jnp.maximum(union, 1e-8)
        sup = (iou > _IOU_TH) & (rows != i) & can
        keep_ref[...] = jnp.where(sup, 0.0, keep)
        return 0

    lax.fori_loop(0, _PRE, body, 0)
    sc = sc_ref[...]
    out_ref[...] = jnp.where((keep_ref[...] > 0.0) & (sc > _SCORE_TH), sc, -1.0)


def _nms(y1l, x1l, y2l, x2l, scl):
    return pl.pallas_call(
        _nms_body,
        out_shape=jax.ShapeDtypeStruct((_PREPAD, _LANES), jnp.float32),
        scratch_shapes=[
            pltpu.VMEM((_PREPAD, _LANES), jnp.float32),
            pltpu.VMEM((_PREPAD, _LANES), jnp.float32),
        ],
    )(y1l, x1l, y2l, x2l, scl)


def _to_lane(a, pad_val):
    a = a.reshape(_INST, _PRE).T
    return jnp.pad(a, ((0, _PREPAD - _PRE), (0, _LANES - _INST)), constant_values=pad_val)


def kernel(pred_deltas, pred_label_probs, prior_boxes):
    boxes_t, scores_t = _decode(pred_deltas, pred_label_probs, prior_boxes)
    scores21 = scores_t[:, :_C, :]
    ts, idx = lax.top_k(scores21, _PRE)
    bsel = jnp.take_along_axis(boxes_t[:, :, None, :], idx[:, None, :, :], axis=3)
    comps = [bsel[:, k] for k in range(4)]
    y1l, x1l, y2l, x2l = [_to_lane(c, 0.0) for c in comps]
    scl = _to_lane(ts, -1.0)
    kept = _nms(y1l, x1l, y2l, x2l, scl)
    kept600 = kept[:_PRE, :_INST].T.reshape(_B, _C, _PRE)
    sel_scores, sidx = lax.top_k(kept600, _MAXT)
    selc = [jnp.take_along_axis(c, sidx, axis=2) for c in comps]
    flat_scores = sel_scores.reshape(_B, _C * _MAXT)
    fs, fidx = lax.top_k(flat_scores, _MAXT)
    fcomp = [jnp.take_along_axis(c.reshape(_B, -1), fidx, axis=1) for c in selc]
    fcls = jnp.take_along_axis(
        jnp.broadcast_to(jnp.arange(_C, dtype=jnp.int32)[None, :, None], (_B, _C, _MAXT)).reshape(_B, -1),
        fidx,
        axis=1,
    )
    ok = fs > 0.0
    final_scores = jnp.where(ok, fs, 0.0)
    final_boxes = jnp.where(ok[..., None], jnp.stack(fcomp, axis=-1), 0.0)
    final_labels = jnp.where(ok, fcls, 0).astype(jnp.float32)
    return final_boxes, final_labels, final_scores


# speedup vs baseline: 9.0647x; 1.0435x over previous
import functools

import jax
import jax.numpy as jnp
from jax import lax
from jax.experimental import pallas as pl
from jax.experimental.pallas import tpu as pltpu

_N = 20000
_NPAD = 20480
_NBLK = 10240
_C = 21
_CPAD = 24
_B = 8
_PRE = 600
_PREPAD = 640
_INST = _B * _C
_LANES = 256
_MAXT = 200
_SCORE_TH = 0.5
_IOU_TH = 0.5


def _decode_body(deltas_ref, priors_ref, probs_ref, boxes_ref, scores_ref):
    d = deltas_ref[0]
    p = priors_ref[...]
    pw = p[3:4] - p[1:2]
    ph = p[2:3] - p[0:1]
    pcx = p[1:2] + 0.5 * pw
    pcy = p[0:1] + 0.5 * ph
    bw = jnp.exp(d[3:4] * 0.2) * pw
    bh = jnp.exp(d[2:3] * 0.2) * ph
    bcx = (d[1:2] * 0.1) * pw + pcx
    bcy = (d[0:1] * 0.1) * ph + pcy
    y1 = bcy - 0.5 * bh
    x1 = bcx - 0.5 * bw
    y2 = bh + y1
    x2 = bw + x1
    boxes_ref[0] = jnp.clip(jnp.concatenate([y1, x1, y2, x2], axis=0), 0.0, 1.0)
    pr = probs_ref[0]
    mx = jnp.max(pr, axis=0, keepdims=True)
    nonbg = pr[0:1] < mx
    scores_ref[0] = jnp.where(nonbg, pr, 0.0)


def _decode(pred_deltas, pred_label_probs, prior_boxes):
    deltas_t = jnp.pad(jnp.swapaxes(pred_deltas, 1, 2), ((0, 0), (0, 0), (0, _NPAD - _N)))
    priors_t = jnp.pad(prior_boxes.T, ((0, 0), (0, _NPAD - _N)))
    probs_t = jnp.pad(
        jnp.swapaxes(pred_label_probs, 1, 2),
        ((0, 0), (0, _CPAD - _C), (0, _NPAD - _N)),
        constant_values=-1.0,
    )
    grid = (_B, _NPAD // _NBLK)
    boxes_t, scores_t = pl.pallas_call(
        _decode_body,
        grid=grid,
        in_specs=[
            pl.BlockSpec((1, 4, _NBLK), lambda b, n: (b, 0, n)),
            pl.BlockSpec((4, _NBLK), lambda b, n: (0, n)),
            pl.BlockSpec((1, _CPAD, _NBLK), lambda b, n: (b, 0, n)),
        ],
        out_specs=[
            pl.BlockSpec((1, 4, _NBLK), lambda b, n: (b, 0, n)),
            pl.BlockSpec((1, _CPAD, _NBLK), lambda b, n: (b, 0, n)),
        ],
        out_shape=[
            jax.ShapeDtypeStruct((_B, 4, _NPAD), jnp.float32),
            jax.ShapeDtypeStruct((_B, _CPAD, _NPAD), jnp.float32),
        ],
    )(deltas_t, priors_t, probs_t)
    return boxes_t, scores_t


def _nms_body(y1_ref, x1_ref, y2_ref, x2_ref, sc_ref, out_ref, area_ref, keep_ref):
    y1 = y1_ref[...]
    x1 = x1_ref[...]
    y2 = y2_ref[...]
    x2 = x2_ref[...]
    area_ref[...] = (y2 - y1) * (x2 - x1)
    keep_ref[...] = jnp.ones_like(y1)

    _BK = 64
    for b0 in range(0, _PRE, _BK):
        nsteps = min(_BK, _PRE - b0)
        suf = _PREPAD - b0
        sy1 = y1_ref[b0:, :]
        sx1 = x1_ref[b0:, :]
        sy2 = y2_ref[b0:, :]
        sx2 = x2_ref[b0:, :]
        sarea = area_ref[b0:, :]
        rows = lax.broadcasted_iota(jnp.int32, (suf, _LANES), 0) + b0

        def body(i, _, sy1=sy1, sx1=sx1, sy2=sy2, sx2=sx2, sarea=sarea, rows=rows, b0=b0, suf=suf):
            ry1 = y1_ref[pl.ds(i, 1), :]
            rx1 = x1_ref[pl.ds(i, 1), :]
            ry2 = y2_ref[pl.ds(i, 1), :]
            rx2 = x2_ref[pl.ds(i, 1), :]
            rsc = sc_ref[pl.ds(i, 1), :]
            rkeep = keep_ref[pl.ds(i, 1), :]
            rarea = area_ref[pl.ds(i, 1), :]
            can = (rkeep > 0.0) & (rsc > _SCORE_TH)
            ih = jnp.maximum(jnp.minimum(sy2, ry2) - jnp.maximum(sy1, ry1), 0.0)
            iw = jnp.maximum(jnp.minimum(sx2, rx2) - jnp.maximum(sx1, rx1), 0.0)
            inter = ih * iw
            union = sarea + rarea - inter
            iou = inter / jnp.maximum(union, 1e-8)
            sup = (iou > _IOU_TH) & (rows != i) & can
            keep_ref[b0:, :] = jnp.where(sup, 0.0, keep_ref[b0:, :])
            return 0

        lax.fori_loop(b0, b0 + nsteps, body, 0)
    sc = sc_ref[...]
    out_ref[...] = jnp.where((keep_ref[...] > 0.0) & (sc > _SCORE_TH), sc, -1.0)


def _nms(y1l, x1l, y2l, x2l, scl):
    return pl.pallas_call(
        _nms_body,
        out_shape=jax.ShapeDtypeStruct((_PREPAD, _LANES), jnp.float32),
        scratch_shapes=[
            pltpu.VMEM((_PREPAD, _LANES), jnp.float32),
            pltpu.VMEM((_PREPAD, _LANES), jnp.float32),
        ],
    )(y1l, x1l, y2l, x2l, scl)


def _to_lane(a, pad_val):
    a = a.reshape(_INST, _PRE).T
    return jnp.pad(a, ((0, _PREPAD - _PRE), (0, _LANES - _INST)), constant_values=pad_val)


def kernel(pred_deltas, pred_label_probs, prior_boxes):
    boxes_t, scores_t = _decode(pred_deltas, pred_label_probs, prior_boxes)
    scores21 = scores_t[:, :_C, :]
    ts, idx = lax.top_k(scores21, _PRE)
    bsel = jnp.take_along_axis(boxes_t[:, :, None, :], idx[:, None, :, :], axis=3)
    comps = [bsel[:, k] for k in range(4)]
    y1l, x1l, y2l, x2l = [_to_lane(c, 0.0) for c in comps]
    scl = _to_lane(ts, -1.0)
    kept = _nms(y1l, x1l, y2l, x2l, scl)
    kept600 = kept[:_PRE, :_INST].T.reshape(_B, _C, _PRE)
    sel_scores, sidx = lax.top_k(kept600, _MAXT)
    selc = [jnp.take_along_axis(c, sidx, axis=2) for c in comps]
    flat_scores = sel_scores.reshape(_B, _C * _MAXT)
    fs, fidx = lax.top_k(flat_scores, _MAXT)
    fcomp = [jnp.take_along_axis(c.reshape(_B, -1), fidx, axis=1) for c in selc]
    fcls = jnp.take_along_axis(
        jnp.broadcast_to(jnp.arange(_C, dtype=jnp.int32)[None, :, None], (_B, _C, _MAXT)).reshape(_B, -1),
        fidx,
        axis=1,
    )
    ok = fs > 0.0
    final_scores = jnp.where(ok, fs, 0.0)
    final_boxes = jnp.where(ok[..., None], jnp.stack(fcomp, axis=-1), 0.0)
    final_labels = jnp.where(ok, fcls, 0).astype(jnp.float32)
    return final_boxes, final_labels, final_scores
